# baseline (speedup 1.0000x reference)
import functools

import jax
import jax.numpy as jnp
from jax import lax
from jax.experimental import pallas as pl
from jax.experimental.pallas import tpu as pltpu
from jax.experimental.pallas import tpu_sc as plsc

N = 2048
E = 32768
H1 = 512
H2 = 128

NC, NS = 2, 16
EPT = E // NS
NCHUNK = EPT // 128
HALF = N // 2
COLW = 512
NPASS = N // COLW
PELEMS = HALF * COLW
PSTRIPE = PELEMS // NS


def _build_adjacency(src, dst, w):
    mesh = plsc.VectorSubcoreMesh(
        core_axis_name="c", subcore_axis_name="s",
        num_cores=NC, num_subcores=NS)

    @functools.partial(
        pl.kernel,
        out_type=[
            jax.ShapeDtypeStruct((2 * NPASS * PELEMS,), jnp.float32),
            jax.ShapeDtypeStruct((N,), jnp.float32),
        ],
        mesh=mesh,
        scratch_types=[
            pltpu.VMEM((EPT,), jnp.int32),
            pltpu.VMEM((EPT,), jnp.int32),
            pltpu.VMEM((EPT,), jnp.float32),
            pltpu.VMEM((NPASS * NCHUNK, 128), jnp.int32),
            pltpu.VMEM((NPASS * NCHUNK, 128), jnp.float32),
            pltpu.VMEM((NCHUNK, 128), jnp.int32),
            pltpu.VMEM((NCHUNK, 128), jnp.float32),
            pltpu.VMEM((8192,), jnp.float32),
            pltpu.VMEM((128,), jnp.float32),
            pltpu.VMEM_SHARED((PELEMS,), jnp.float32),
            pltpu.VMEM_SHARED((PELEMS,), jnp.float32),
            pltpu.VMEM_SHARED((HALF,), jnp.float32),
            pltpu.SemaphoreType.DMA,
        ],
    )
    def adj_kernel(src_h, dst_h, w_h, a_out, deg_out,
                   src_v, dst_v, w_v, idx_v, val_v, didx_v, dval_v,
                   zeros_v, junk_v, sh_a0, sh_a1, sh_d, sem):
        bufs = (sh_a0, sh_a1)
        cid = lax.axis_index("c")
        sid = lax.axis_index("s")
        base_e = sid * EPT
        loads = [
            pltpu.async_copy(src_h.at[pl.ds(base_e, EPT)], src_v, sem),
            pltpu.async_copy(dst_h.at[pl.ds(base_e, EPT)], dst_v, sem),
            pltpu.async_copy(w_h.at[pl.ds(base_e, EPT)], w_v, sem),
        ]

        def zero_body(i, carry):
            zeros_v[pl.ds(i * 16, 16)] = jnp.zeros((16,), jnp.float32)
            return carry
        lax.fori_loop(0, 8192 // 16, zero_body, 0)

        def zero_stripe(buf):
            descs = [
                pltpu.async_copy(
                    zeros_v, buf.at[pl.ds(sid * PSTRIPE + jz * 8192, 8192)], sem)
                for jz in range(PSTRIPE // 8192)]
            for d in descs:
                d.wait()

        zero_stripe(sh_a0)
        zero_stripe(sh_a1)

        @pl.when(sid == 0)
        def _zero_deg():
            pltpu.sync_copy(zeros_v.at[pl.ds(0, HALF)], sh_d)

        for d in loads:
            d.wait()
        plsc.subcore_barrier()

        row0 = cid * HALF
        for k in range(NPASS):
            buf = bufs[k % 2]
            col0 = k * COLW

            def chunk_body(j, carry):
                for v in range(8):
                    e0 = j * 128 + v * 16
                    s_ = src_v[pl.ds(e0, 16)]
                    d_ = dst_v[pl.ds(e0, 16)]
                    wv = w_v[pl.ds(e0, 16)]
                    dl = d_ - row0
                    sl = s_ - col0
                    in_row = (d_ >= row0) & (d_ < row0 + HALF)
                    m = in_row & (s_ >= col0) & (s_ < col0 + COLW)
                    idx_v[k * NCHUNK + j, pl.ds(v * 16, 16)] = jnp.where(
                        m, dl * COLW + sl, 0)
                    val_v[k * NCHUNK + j, pl.ds(v * 16, 16)] = jnp.where(m, wv, 0.0)
                    if k == 0:
                        didx_v[j, pl.ds(v * 16, 16)] = jnp.where(in_row, dl, 0)
                        dval_v[j, pl.ds(v * 16, 16)] = jnp.where(in_row, 1.0, 0.0)
                return carry
            lax.fori_loop(0, NCHUNK, chunk_body, 0)

            adds = [
                pltpu.async_copy(val_v.at[k * NCHUNK + j],
                                 buf.at[idx_v.at[k * NCHUNK + j]], sem, add=True)
                for j in range(NCHUNK)]
            if k == 0:
                adds += [
                    pltpu.async_copy(dval_v.at[j], sh_d.at[didx_v.at[j]], sem,
                                     add=True)
                    for j in range(NCHUNK)]
            for d in adds:
                d.wait()

            flushes = [
                pltpu.async_copy(buf.at[idx_v.at[k * NCHUNK + j]], junk_v, sem)
                for j in range(NCHUNK)]
            for d in flushes:
                d.wait()
            plsc.subcore_barrier()

            qbase = (cid * NPASS + k) * PELEMS + sid * PSTRIPE
            pltpu.sync_copy(buf.at[pl.ds(sid * PSTRIPE, PSTRIPE)],
                            a_out.at[pl.ds(qbase, PSTRIPE)])
            if k + 2 < NPASS:
                zero_stripe(buf)
            plsc.subcore_barrier()

        @pl.when(sid == 0)
        def _write_deg():
            pltpu.sync_copy(sh_d, deg_out.at[pl.ds(cid * HALF, HALF)])

    return adj_kernel(src, dst, w)


_BM = 256


def _mm_rt_kernel(x_ref, w_ref, o_ref):
    o_ref[...] = lax.dot_general(
        x_ref[...], w_ref[...], (((1,), (1,)), ((), ())),
        preferred_element_type=jnp.float32, precision=lax.Precision.HIGHEST)


def _matmul_rt(x, w):
    m, k = x.shape
    n = w.shape[0]
    return pl.pallas_call(
        _mm_rt_kernel,
        grid=(m // _BM,),
        in_specs=[
            pl.BlockSpec((_BM, k), lambda i: (i, 0)),
            pl.BlockSpec((n, k), lambda i: (0, 0)),
        ],
        out_specs=pl.BlockSpec((_BM, n), lambda i: (i, 0)),
        out_shape=jax.ShapeDtypeStruct((m, n), jnp.float32),
    )(x, w)


def _conv_layer(a_q, y, r, b, w2=None, relu=False):
    m, h = y.shape
    nblk_half = HALF // _BM

    def body(a_ref, yfull_ref, yblk_ref, r_ref, b_ref, *rest):
        if w2 is not None:
            w2_ref, o_ref = rest
        else:
            (o_ref,) = rest
        acc = lax.dot_general(
            a_ref[0, 0], yfull_ref[pl.ds(0, COLW), :],
            (((1,), (0,)), ((), ())), preferred_element_type=jnp.float32,
            precision=lax.Precision.HIGHEST)
        for s in range(1, NPASS):
            acc += lax.dot_general(
                a_ref[0, s], yfull_ref[pl.ds(s * COLW, COLW), :],
                (((1,), (0,)), ((), ())), preferred_element_type=jnp.float32,
                precision=lax.Precision.HIGHEST)
        h_ = (acc + yblk_ref[...]) * r_ref[...] + b_ref[...]
        if relu:
            h_ = jnp.maximum(h_, 0.0)
        if w2 is not None:
            h_ = lax.dot_general(
                h_, w2_ref[...], (((1,), (1,)), ((), ())),
                preferred_element_type=jnp.float32,
                precision=lax.Precision.HIGHEST)
        o_ref[...] = h_

    out_n = w2.shape[0] if w2 is not None else h
    in_specs = [
        pl.BlockSpec((1, NPASS, _BM, COLW),
                     lambda i: (i // nblk_half, 0, i % nblk_half, 0)),
        pl.BlockSpec((m, h), lambda i: (0, 0)),
        pl.BlockSpec((_BM, h), lambda i: (i, 0)),
        pl.BlockSpec((_BM, 1), lambda i: (i, 0)),
        pl.BlockSpec((1, h), lambda i: (0, 0)),
    ]
    args = [a_q, y, y, r, b.reshape(1, h)]
    if w2 is not None:
        in_specs.append(pl.BlockSpec((out_n, h), lambda i: (0, 0)))
        args.append(w2)
    return pl.pallas_call(
        body,
        grid=(m // _BM,),
        in_specs=in_specs,
        out_specs=pl.BlockSpec((_BM, out_n), lambda i: (i, 0)),
        out_shape=jax.ShapeDtypeStruct((m, out_n), jnp.float32),
    )(*args)


def _gram_kernel(zi_ref, zj_ref, o_ref):
    o_ref[...] = lax.dot_general(
        zi_ref[...], zj_ref[...], (((1,), (1,)), ((), ())),
        preferred_element_type=jnp.float32, precision=lax.Precision.HIGHEST)


def _gram(z):
    m, h = z.shape
    nb = m // _BM
    return pl.pallas_call(
        _gram_kernel,
        grid=(nb, nb),
        in_specs=[
            pl.BlockSpec((_BM, h), lambda i, j: (i, 0)),
            pl.BlockSpec((_BM, h), lambda i, j: (j, 0)),
        ],
        out_specs=pl.BlockSpec((_BM, _BM), lambda i, j: (i, j)),
        out_shape=jax.ShapeDtypeStruct((m, m), jnp.float32),
    )(z, z)


def kernel(edge_index, edge_weight, feat, W1, b1, W2, b2):
    src = edge_index[0]
    dst = edge_index[1]
    a_flat, deg = _build_adjacency(src, dst, edge_weight)
    a_q = a_flat.reshape(2, NPASS, HALF, COLW)
    r = (1.0 / (deg + 1.0)).reshape(N, 1)

    y1 = _matmul_rt(feat, W1)
    y2 = _conv_layer(a_q, y1, r, b1, w2=W2, relu=True)
    z = _conv_layer(a_q, y2, r, b2)
    adj_rec = _gram(z)
    return (z, adj_rec)


# speedup vs baseline: 1.2196x; 1.2196x over previous
import functools

import jax
import jax.numpy as jnp
from jax import lax
from jax.experimental import pallas as pl
from jax.experimental.pallas import tpu as pltpu
from jax.experimental.pallas import tpu_sc as plsc

N = 2048
E = 32768
H1 = 512
H2 = 128

NC, NS = 2, 16
EPT = E // NS
NCHUNK = EPT // 128
HALF = N // 2
COLW = 512
NPASS = N // COLW
PELEMS = HALF * COLW
PSTRIPE = PELEMS // NS


def _build_adjacency(src, dst, w):
    mesh = plsc.VectorSubcoreMesh(
        core_axis_name="c", subcore_axis_name="s",
        num_cores=NC, num_subcores=NS)

    @functools.partial(
        pl.kernel,
        out_type=[
            jax.ShapeDtypeStruct((2 * NPASS * PELEMS,), jnp.float32),
            jax.ShapeDtypeStruct((N,), jnp.float32),
        ],
        mesh=mesh,
        scratch_types=[
            pltpu.VMEM((EPT,), jnp.int32),
            pltpu.VMEM((EPT,), jnp.int32),
            pltpu.VMEM((EPT,), jnp.float32),
            pltpu.VMEM((NPASS * NCHUNK, 128), jnp.int32),
            pltpu.VMEM((NPASS * NCHUNK, 128), jnp.float32),
            pltpu.VMEM((NCHUNK, 128), jnp.int32),
            pltpu.VMEM((NCHUNK, 128), jnp.float32),
            pltpu.VMEM((8192,), jnp.float32),
            pltpu.VMEM((128,), jnp.float32),
            pltpu.VMEM_SHARED((PELEMS,), jnp.float32),
            pltpu.VMEM_SHARED((PELEMS,), jnp.float32),
            pltpu.VMEM_SHARED((HALF,), jnp.float32),
            pltpu.SemaphoreType.DMA,
        ],
    )
    def adj_kernel(src_h, dst_h, w_h, a_out, deg_out,
                   src_v, dst_v, w_v, idx_v, val_v, didx_v, dval_v,
                   zeros_v, junk_v, sh_a0, sh_a1, sh_d, sem):
        bufs = (sh_a0, sh_a1)
        cid = lax.axis_index("c")
        sid = lax.axis_index("s")
        base_e = sid * EPT
        loads = [
            pltpu.async_copy(src_h.at[pl.ds(base_e, EPT)], src_v, sem),
            pltpu.async_copy(dst_h.at[pl.ds(base_e, EPT)], dst_v, sem),
            pltpu.async_copy(w_h.at[pl.ds(base_e, EPT)], w_v, sem),
        ]

        def zero_body(i, carry):
            zeros_v[pl.ds(i * 16, 16)] = jnp.zeros((16,), jnp.float32)
            return carry
        lax.fori_loop(0, 8192 // 16, zero_body, 0)

        def zero_stripe(buf):
            descs = [
                pltpu.async_copy(
                    zeros_v, buf.at[pl.ds(sid * PSTRIPE + jz * 8192, 8192)], sem)
                for jz in range(PSTRIPE // 8192)]
            for d in descs:
                d.wait()

        zero_stripe(sh_a0)
        zero_stripe(sh_a1)

        @pl.when(sid == 0)
        def _zero_deg():
            pltpu.sync_copy(zeros_v.at[pl.ds(0, HALF)], sh_d)

        for d in loads:
            d.wait()
        plsc.subcore_barrier()

        row0 = cid * HALF
        for k in range(NPASS):
            buf = bufs[k % 2]
            col0 = k * COLW

            def chunk_body(j, carry):
                for v in range(8):
                    e0 = j * 128 + v * 16
                    s_ = src_v[pl.ds(e0, 16)]
                    d_ = dst_v[pl.ds(e0, 16)]
                    wv = w_v[pl.ds(e0, 16)]
                    dl = d_ - row0
                    sl = s_ - col0
                    in_row = (d_ >= row0) & (d_ < row0 + HALF)
                    m = in_row & (s_ >= col0) & (s_ < col0 + COLW)
                    idx_v[k * NCHUNK + j, pl.ds(v * 16, 16)] = jnp.where(
                        m, dl * COLW + sl, 0)
                    val_v[k * NCHUNK + j, pl.ds(v * 16, 16)] = jnp.where(m, wv, 0.0)
                    if k == 0:
                        didx_v[j, pl.ds(v * 16, 16)] = jnp.where(in_row, dl, 0)
                        dval_v[j, pl.ds(v * 16, 16)] = jnp.where(in_row, 1.0, 0.0)
                return carry
            lax.fori_loop(0, NCHUNK, chunk_body, 0)

            adds = [
                pltpu.async_copy(val_v.at[k * NCHUNK + j],
                                 buf.at[idx_v.at[k * NCHUNK + j]], sem, add=True)
                for j in range(NCHUNK)]
            if k == 0:
                adds += [
                    pltpu.async_copy(dval_v.at[j], sh_d.at[didx_v.at[j]], sem,
                                     add=True)
                    for j in range(NCHUNK)]
            for d in adds:
                d.wait()

            plsc.subcore_barrier()

            if k >= 1:
                kp = k - 1
                pbuf = bufs[kp % 2]
                qbase = (cid * NPASS + kp) * PELEMS + sid * PSTRIPE
                pltpu.sync_copy(pbuf.at[pl.ds(sid * PSTRIPE, PSTRIPE)],
                                a_out.at[pl.ds(qbase, PSTRIPE)])
                if kp + 2 < NPASS:
                    zero_stripe(pbuf)
            plsc.subcore_barrier()

        kp = NPASS - 1
        pbuf = bufs[kp % 2]
        qbase = (cid * NPASS + kp) * PELEMS + sid * PSTRIPE
        pltpu.sync_copy(pbuf.at[pl.ds(sid * PSTRIPE, PSTRIPE)],
                        a_out.at[pl.ds(qbase, PSTRIPE)])

        @pl.when(sid == 0)
        def _write_deg():
            pltpu.sync_copy(sh_d, deg_out.at[pl.ds(cid * HALF, HALF)])

    return adj_kernel(src, dst, w)


_BM = 256


def _mm_rt_kernel(x_ref, w_ref, o_ref):
    o_ref[...] = lax.dot_general(
        x_ref[...], w_ref[...], (((1,), (1,)), ((), ())),
        preferred_element_type=jnp.float32, precision=lax.Precision.HIGHEST)


def _matmul_rt(x, w):
    m, k = x.shape
    n = w.shape[0]
    return pl.pallas_call(
        _mm_rt_kernel,
        grid=(m // _BM,),
        in_specs=[
            pl.BlockSpec((_BM, k), lambda i: (i, 0)),
            pl.BlockSpec((n, k), lambda i: (0, 0)),
        ],
        out_specs=pl.BlockSpec((_BM, n), lambda i: (i, 0)),
        out_shape=jax.ShapeDtypeStruct((m, n), jnp.float32),
    )(x, w)


def _conv_layer(a_q, y, r, b, w2=None, relu=False):
    m, h = y.shape
    nblk_half = HALF // _BM

    def body(a_ref, yfull_ref, yblk_ref, r_ref, b_ref, *rest):
        if w2 is not None:
            w2_ref, o_ref = rest
        else:
            (o_ref,) = rest
        acc = lax.dot_general(
            a_ref[0, 0], yfull_ref[pl.ds(0, COLW), :],
            (((1,), (0,)), ((), ())), preferred_element_type=jnp.float32,
            precision=lax.Precision.HIGHEST)
        for s in range(1, NPASS):
            acc += lax.dot_general(
                a_ref[0, s], yfull_ref[pl.ds(s * COLW, COLW), :],
                (((1,), (0,)), ((), ())), preferred_element_type=jnp.float32,
                precision=lax.Precision.HIGHEST)
        h_ = (acc + yblk_ref[...]) * r_ref[...] + b_ref[...]
        if relu:
            h_ = jnp.maximum(h_, 0.0)
        if w2 is not None:
            h_ = lax.dot_general(
                h_, w2_ref[...], (((1,), (1,)), ((), ())),
                preferred_element_type=jnp.float32,
                precision=lax.Precision.HIGHEST)
        o_ref[...] = h_

    out_n = w2.shape[0] if w2 is not None else h
    in_specs = [
        pl.BlockSpec((1, NPASS, _BM, COLW),
                     lambda i: (i // nblk_half, 0, i % nblk_half, 0)),
        pl.BlockSpec((m, h), lambda i: (0, 0)),
        pl.BlockSpec((_BM, h), lambda i: (i, 0)),
        pl.BlockSpec((_BM, 1), lambda i: (i, 0)),
        pl.BlockSpec((1, h), lambda i: (0, 0)),
    ]
    args = [a_q, y, y, r, b.reshape(1, h)]
    if w2 is not None:
        in_specs.append(pl.BlockSpec((out_n, h), lambda i: (0, 0)))
        args.append(w2)
    return pl.pallas_call(
        body,
        grid=(m // _BM,),
        in_specs=in_specs,
        out_specs=pl.BlockSpec((_BM, out_n), lambda i: (i, 0)),
        out_shape=jax.ShapeDtypeStruct((m, out_n), jnp.float32),
    )(*args)


def _gram_kernel(zi_ref, zj_ref, o_ref):
    o_ref[...] = lax.dot_general(
        zi_ref[...], zj_ref[...], (((1,), (1,)), ((), ())),
        preferred_element_type=jnp.float32, precision=lax.Precision.HIGHEST)


def _gram(z):
    m, h = z.shape
    nb = m // _BM
    return pl.pallas_call(
        _gram_kernel,
        grid=(nb, nb),
        in_specs=[
            pl.BlockSpec((_BM, h), lambda i, j: (i, 0)),
            pl.BlockSpec((_BM, h), lambda i, j: (j, 0)),
        ],
        out_specs=pl.BlockSpec((_BM, _BM), lambda i, j: (i, j)),
        out_shape=jax.ShapeDtypeStruct((m, m), jnp.float32),
    )(z, z)


def kernel(edge_index, edge_weight, feat, W1, b1, W2, b2):
    src = edge_index[0]
    dst = edge_index[1]
    a_flat, deg = _build_adjacency(src, dst, edge_weight)
    a_q = a_flat.reshape(2, NPASS, HALF, COLW)
    r = (1.0 / (deg + 1.0)).reshape(N, 1)

    y1 = _matmul_rt(feat, W1)
    y2 = _conv_layer(a_q, y1, r, b1, w2=W2, relu=True)
    z = _conv_layer(a_q, y2, r, b2)
    adj_rec = _gram(z)
    return (z, adj_rec)


# speedup vs baseline: 2.1535x; 1.7657x over previous
import functools

import jax
import jax.numpy as jnp
from jax import lax
from jax.experimental import pallas as pl
from jax.experimental.pallas import tpu as pltpu
from jax.experimental.pallas import tpu_sc as plsc

N = 2048
E = 32768
H1 = 512
H2 = 128

NC, NS = 2, 16
EPT = E // NS
NCHUNK = EPT // 128
HALF = N // 2
COLW = 512
NPASS = N // COLW
PELEMS = HALF * COLW
PSTRIPE = PELEMS // NS


def _build_adjacency(src, dst, w):
    mesh = plsc.VectorSubcoreMesh(
        core_axis_name="c", subcore_axis_name="s",
        num_cores=NC, num_subcores=NS)

    @functools.partial(
        pl.kernel,
        out_type=[
            jax.ShapeDtypeStruct((2 * NPASS * PELEMS,), jnp.float32),
            jax.ShapeDtypeStruct((N,), jnp.float32),
        ],
        mesh=mesh,
        scratch_types=[
            pltpu.VMEM((EPT,), jnp.int32),
            pltpu.VMEM((EPT,), jnp.int32),
            pltpu.VMEM((EPT,), jnp.float32),
            [pltpu.VMEM((EPT,), jnp.int32) for _ in range(NPASS)],
            [pltpu.VMEM((EPT,), jnp.float32) for _ in range(NPASS)],
            pltpu.VMEM((EPT,), jnp.int32),
            pltpu.VMEM((EPT,), jnp.float32),
            pltpu.VMEM((8192,), jnp.float32),
            pltpu.VMEM((128,), jnp.float32),
            pltpu.VMEM_SHARED((PELEMS,), jnp.float32),
            pltpu.VMEM_SHARED((PELEMS,), jnp.float32),
            pltpu.VMEM_SHARED((HALF,), jnp.float32),
            pltpu.SemaphoreType.DMA,
        ],
    )
    def adj_kernel(src_h, dst_h, w_h, a_out, deg_out,
                   src_v, dst_v, w_v, idx_v, val_v, didx_v, dval_v,
                   zeros_v, junk_v, sh_a0, sh_a1, sh_d, sem):
        bufs = (sh_a0, sh_a1)
        cid = lax.axis_index("c")
        sid = lax.axis_index("s")
        base_e = sid * EPT
        loads = [
            pltpu.async_copy(src_h.at[pl.ds(base_e, EPT)], src_v, sem),
            pltpu.async_copy(dst_h.at[pl.ds(base_e, EPT)], dst_v, sem),
            pltpu.async_copy(w_h.at[pl.ds(base_e, EPT)], w_v, sem),
        ]

        def zero_body(i, carry):
            zeros_v[pl.ds(i * 16, 16)] = jnp.zeros((16,), jnp.float32)
            return carry
        lax.fori_loop(0, 8192 // 16, zero_body, 0)

        def zero_stripe(buf):
            descs = [
                pltpu.async_copy(
                    zeros_v, buf.at[pl.ds(sid * PSTRIPE + jz * 8192, 8192)], sem)
                for jz in range(PSTRIPE // 8192)]
            for d in descs:
                d.wait()

        zero_stripe(sh_a0)
        zero_stripe(sh_a1)

        @pl.when(sid == 0)
        def _zero_deg():
            pltpu.sync_copy(zeros_v.at[pl.ds(0, HALF)], sh_d)

        for d in loads:
            d.wait()
        plsc.subcore_barrier()

        row0 = cid * HALF
        for k in range(NPASS):
            buf = bufs[k % 2]
            col0 = k * COLW

            def chunk_body(j, carry):
                for v in range(8):
                    e0 = j * 128 + v * 16
                    s_ = src_v[pl.ds(e0, 16)]
                    d_ = dst_v[pl.ds(e0, 16)]
                    wv = w_v[pl.ds(e0, 16)]
                    dl = d_ - row0
                    sl = s_ - col0
                    in_row = (d_ >= row0) & (d_ < row0 + HALF)
                    m = in_row & (s_ >= col0) & (s_ < col0 + COLW)
                    dummy = lax.iota(jnp.int32, 16) + e0
                    idx_v[k][pl.ds(e0, 16)] = jnp.where(m, dl * COLW + sl, dummy)
                    val_v[k][pl.ds(e0, 16)] = jnp.where(m, wv, 0.0)
                    if k == 0:
                        didx_v[pl.ds(e0, 16)] = jnp.where(
                            in_row, dl, dummy & (HALF - 1))
                        dval_v[pl.ds(e0, 16)] = jnp.where(in_row, 1.0, 0.0)
                return carry
            lax.fori_loop(0, NCHUNK, chunk_body, 0)

            adds = [pltpu.async_copy(val_v[k], buf.at[idx_v[k]], sem,
                                     add=True)]
            if k == 0:
                adds.append(
                    pltpu.async_copy(dval_v, sh_d.at[didx_v], sem, add=True))
            for d in adds:
                d.wait()

            plsc.subcore_barrier()

            if k >= 1:
                kp = k - 1
                pbuf = bufs[kp % 2]
                qbase = (cid * NPASS + kp) * PELEMS + sid * PSTRIPE
                pltpu.sync_copy(pbuf.at[pl.ds(sid * PSTRIPE, PSTRIPE)],
                                a_out.at[pl.ds(qbase, PSTRIPE)])
                if kp + 2 < NPASS:
                    zero_stripe(pbuf)
            plsc.subcore_barrier()

        kp = NPASS - 1
        pbuf = bufs[kp % 2]
        qbase = (cid * NPASS + kp) * PELEMS + sid * PSTRIPE
        pltpu.sync_copy(pbuf.at[pl.ds(sid * PSTRIPE, PSTRIPE)],
                        a_out.at[pl.ds(qbase, PSTRIPE)])

        @pl.when(sid == 0)
        def _write_deg():
            pltpu.sync_copy(sh_d, deg_out.at[pl.ds(cid * HALF, HALF)])

    return adj_kernel(src, dst, w)


_BM = 256


def _mm_rt_kernel(x_ref, w_ref, o_ref):
    o_ref[...] = lax.dot_general(
        x_ref[...], w_ref[...], (((1,), (1,)), ((), ())),
        preferred_element_type=jnp.float32, precision=lax.Precision.HIGHEST)


def _matmul_rt(x, w):
    m, k = x.shape
    n = w.shape[0]
    return pl.pallas_call(
        _mm_rt_kernel,
        grid=(m // _BM,),
        in_specs=[
            pl.BlockSpec((_BM, k), lambda i: (i, 0)),
            pl.BlockSpec((n, k), lambda i: (0, 0)),
        ],
        out_specs=pl.BlockSpec((_BM, n), lambda i: (i, 0)),
        out_shape=jax.ShapeDtypeStruct((m, n), jnp.float32),
    )(x, w)


def _conv_layer(a_q, y, r, b, w2=None, relu=False):
    m, h = y.shape
    nblk_half = HALF // _BM

    def body(a_ref, yfull_ref, yblk_ref, r_ref, b_ref, *rest):
        if w2 is not None:
            w2_ref, o_ref = rest
        else:
            (o_ref,) = rest
        acc = lax.dot_general(
            a_ref[0, 0], yfull_ref[pl.ds(0, COLW), :],
            (((1,), (0,)), ((), ())), preferred_element_type=jnp.float32,
            precision=lax.Precision.HIGHEST)
        for s in range(1, NPASS):
            acc += lax.dot_general(
                a_ref[0, s], yfull_ref[pl.ds(s * COLW, COLW), :],
                (((1,), (0,)), ((), ())), preferred_element_type=jnp.float32,
                precision=lax.Precision.HIGHEST)
        h_ = (acc + yblk_ref[...]) * r_ref[...] + b_ref[...]
        if relu:
            h_ = jnp.maximum(h_, 0.0)
        if w2 is not None:
            h_ = lax.dot_general(
                h_, w2_ref[...], (((1,), (1,)), ((), ())),
                preferred_element_type=jnp.float32,
                precision=lax.Precision.HIGHEST)
        o_ref[...] = h_

    out_n = w2.shape[0] if w2 is not None else h
    in_specs = [
        pl.BlockSpec((1, NPASS, _BM, COLW),
                     lambda i: (i // nblk_half, 0, i % nblk_half, 0)),
        pl.BlockSpec((m, h), lambda i: (0, 0)),
        pl.BlockSpec((_BM, h), lambda i: (i, 0)),
        pl.BlockSpec((_BM, 1), lambda i: (i, 0)),
        pl.BlockSpec((1, h), lambda i: (0, 0)),
    ]
    args = [a_q, y, y, r, b.reshape(1, h)]
    if w2 is not None:
        in_specs.append(pl.BlockSpec((out_n, h), lambda i: (0, 0)))
        args.append(w2)
    return pl.pallas_call(
        body,
        grid=(m // _BM,),
        in_specs=in_specs,
        out_specs=pl.BlockSpec((_BM, out_n), lambda i: (i, 0)),
        out_shape=jax.ShapeDtypeStruct((m, out_n), jnp.float32),
    )(*args)


def _gram_kernel(zi_ref, zj_ref, o_ref):
    o_ref[...] = lax.dot_general(
        zi_ref[...], zj_ref[...], (((1,), (1,)), ((), ())),
        preferred_element_type=jnp.float32, precision=lax.Precision.HIGHEST)


def _gram(z):
    m, h = z.shape
    nb = m // _BM
    return pl.pallas_call(
        _gram_kernel,
        grid=(nb, nb),
        in_specs=[
            pl.BlockSpec((_BM, h), lambda i, j: (i, 0)),
            pl.BlockSpec((_BM, h), lambda i, j: (j, 0)),
        ],
        out_specs=pl.BlockSpec((_BM, _BM), lambda i, j: (i, j)),
        out_shape=jax.ShapeDtypeStruct((m, m), jnp.float32),
    )(z, z)


def kernel(edge_index, edge_weight, feat, W1, b1, W2, b2):
    src = edge_index[0]
    dst = edge_index[1]
    a_flat, deg = _build_adjacency(src, dst, edge_weight)
    a_q = a_flat.reshape(2, NPASS, HALF, COLW)
    r = (1.0 / (deg + 1.0)).reshape(N, 1)

    y1 = _matmul_rt(feat, W1)
    y2 = _conv_layer(a_q, y1, r, b1, w2=W2, relu=True)
    z = _conv_layer(a_q, y2, r, b2)
    adj_rec = _gram(z)
    return (z, adj_rec)


# speedup vs baseline: 2.6329x; 1.2226x over previous
import functools

import jax
import jax.numpy as jnp
from jax import lax
from jax.experimental import pallas as pl
from jax.experimental.pallas import tpu as pltpu
from jax.experimental.pallas import tpu_sc as plsc

N = 2048
E = 32768
H1 = 512
H2 = 128

NC, NS = 2, 16
EPT = E // NS
NCHUNK = EPT // 128
HALF = N // 2
COLW = 512
NPASS = N // COLW
PELEMS = HALF * COLW
PSTRIPE = PELEMS // NS


def _build_adjacency(src, dst, w):
    mesh = plsc.VectorSubcoreMesh(
        core_axis_name="c", subcore_axis_name="s",
        num_cores=NC, num_subcores=NS)

    @functools.partial(
        pl.kernel,
        out_type=[
            jax.ShapeDtypeStruct((2 * NPASS * PELEMS,), jnp.float32),
            jax.ShapeDtypeStruct((N,), jnp.float32),
        ],
        mesh=mesh,
        scratch_types=[
            pltpu.VMEM((EPT,), jnp.int32),
            pltpu.VMEM((EPT,), jnp.int32),
            pltpu.VMEM((EPT,), jnp.float32),
            [pltpu.VMEM((EPT,), jnp.int32) for _ in range(NPASS)],
            [pltpu.VMEM((EPT,), jnp.float32) for _ in range(NPASS)],
            pltpu.VMEM((EPT,), jnp.int32),
            pltpu.VMEM((EPT,), jnp.float32),
            pltpu.VMEM((8192,), jnp.float32),
            pltpu.VMEM((128,), jnp.float32),
            pltpu.VMEM_SHARED((PELEMS,), jnp.float32),
            pltpu.VMEM_SHARED((PELEMS,), jnp.float32),
            pltpu.VMEM_SHARED((HALF,), jnp.float32),
            pltpu.SemaphoreType.DMA,
        ],
    )
    def adj_kernel(src_h, dst_h, w_h, a_out, deg_out,
                   src_v, dst_v, w_v, idx_v, val_v, didx_v, dval_v,
                   zeros_v, junk_v, sh_a0, sh_a1, sh_d, sem):
        bufs = (sh_a0, sh_a1)
        cid = lax.axis_index("c")
        sid = lax.axis_index("s")
        base_e = sid * EPT
        loads = [
            pltpu.async_copy(src_h.at[pl.ds(base_e, EPT)], src_v, sem),
            pltpu.async_copy(dst_h.at[pl.ds(base_e, EPT)], dst_v, sem),
            pltpu.async_copy(w_h.at[pl.ds(base_e, EPT)], w_v, sem),
        ]

        def zero_body(i, carry):
            zeros_v[pl.ds(i * 16, 16)] = jnp.zeros((16,), jnp.float32)
            return carry
        lax.fori_loop(0, 8192 // 16, zero_body, 0)

        def zero_stripe(buf):
            descs = [
                pltpu.async_copy(
                    zeros_v, buf.at[pl.ds(sid * PSTRIPE + jz * 8192, 8192)], sem)
                for jz in range(PSTRIPE // 8192)]
            for d in descs:
                d.wait()

        zero_stripe(sh_a0)
        zero_stripe(sh_a1)

        @pl.when(sid == 0)
        def _zero_deg():
            pltpu.sync_copy(zeros_v.at[pl.ds(0, HALF)], sh_d)

        for d in loads:
            d.wait()
        plsc.subcore_barrier()

        row0 = cid * HALF
        for k in range(NPASS):
            buf = bufs[k % 2]
            col0 = k * COLW

            def chunk_body(j, carry):
                for v in range(8):
                    e0 = j * 128 + v * 16
                    s_ = src_v[pl.ds(e0, 16)]
                    d_ = dst_v[pl.ds(e0, 16)]
                    wv = w_v[pl.ds(e0, 16)]
                    dl = d_ - row0
                    sl = s_ - col0
                    in_row = (d_ >= row0) & (d_ < row0 + HALF)
                    m = in_row & (s_ >= col0) & (s_ < col0 + COLW)
                    dummy = lax.iota(jnp.int32, 16) + e0
                    idx_v[k][pl.ds(e0, 16)] = jnp.where(m, dl * COLW + sl, dummy)
                    val_v[k][pl.ds(e0, 16)] = jnp.where(m, wv, 0.0)
                    if k == 0:
                        didx_v[pl.ds(e0, 16)] = jnp.where(
                            in_row, dl, dummy & (HALF - 1))
                        dval_v[pl.ds(e0, 16)] = jnp.where(in_row, 1.0, 0.0)
                return carry
            lax.fori_loop(0, NCHUNK, chunk_body, 0)

            adds = [pltpu.async_copy(val_v[k], buf.at[idx_v[k]], sem,
                                     add=True)]
            if k == 0:
                adds.append(
                    pltpu.async_copy(dval_v, sh_d.at[didx_v], sem, add=True))
            for d in adds:
                d.wait()

            plsc.subcore_barrier()

            if k >= 1:
                kp = k - 1
                pbuf = bufs[kp % 2]
                qbase = (cid * NPASS + kp) * PELEMS + sid * PSTRIPE
                pltpu.sync_copy(pbuf.at[pl.ds(sid * PSTRIPE, PSTRIPE)],
                                a_out.at[pl.ds(qbase, PSTRIPE)])
                if kp + 2 < NPASS:
                    zero_stripe(pbuf)
            plsc.subcore_barrier()

        kp = NPASS - 1
        pbuf = bufs[kp % 2]
        qbase = (cid * NPASS + kp) * PELEMS + sid * PSTRIPE
        pltpu.sync_copy(pbuf.at[pl.ds(sid * PSTRIPE, PSTRIPE)],
                        a_out.at[pl.ds(qbase, PSTRIPE)])

        @pl.when(sid == 0)
        def _write_deg():
            pltpu.sync_copy(sh_d, deg_out.at[pl.ds(cid * HALF, HALF)])

    return adj_kernel(src, dst, w)


_BM = 256


def _split_bf16(x):
    hi = x.astype(jnp.bfloat16)
    lo = (x - hi.astype(jnp.float32)).astype(jnp.bfloat16)
    return hi, lo


def _dot3(x, w, dims):
    xh, xl = _split_bf16(x)
    wh, wl = _split_bf16(w)
    acc = lax.dot_general(xh, wh, dims, preferred_element_type=jnp.float32)
    acc += lax.dot_general(xh, wl, dims, preferred_element_type=jnp.float32)
    acc += lax.dot_general(xl, wh, dims, preferred_element_type=jnp.float32)
    return acc


_DIMS_RT = (((1,), (1,)), ((), ()))
_DIMS_NN = (((1,), (0,)), ((), ()))


def _mm_rt_kernel(x_ref, w_ref, o_ref):
    o_ref[...] = _dot3(x_ref[...], w_ref[...], _DIMS_RT)


def _matmul_rt(x, w):
    m, k = x.shape
    n = w.shape[0]
    return pl.pallas_call(
        _mm_rt_kernel,
        grid=(m // _BM,),
        in_specs=[
            pl.BlockSpec((_BM, k), lambda i: (i, 0)),
            pl.BlockSpec((n, k), lambda i: (0, 0)),
        ],
        out_specs=pl.BlockSpec((_BM, n), lambda i: (i, 0)),
        out_shape=jax.ShapeDtypeStruct((m, n), jnp.float32),
    )(x, w)


def _conv_layer(a_q, y, r, b, w2=None, relu=False):
    m, h = y.shape
    nblk_half = HALF // _BM

    def body(a_ref, yfull_ref, yblk_ref, r_ref, b_ref, *rest):
        if w2 is not None:
            w2_ref, o_ref = rest
        else:
            (o_ref,) = rest
        acc = _dot3(a_ref[0, 0], yfull_ref[pl.ds(0, COLW), :], _DIMS_NN)
        for s in range(1, NPASS):
            acc += _dot3(a_ref[0, s], yfull_ref[pl.ds(s * COLW, COLW), :],
                         _DIMS_NN)
        h_ = (acc + yblk_ref[...]) * r_ref[...] + b_ref[...]
        if relu:
            h_ = jnp.maximum(h_, 0.0)
        if w2 is not None:
            h_ = lax.dot_general(
                h_, w2_ref[...], _DIMS_RT,
                preferred_element_type=jnp.float32)
        o_ref[...] = h_

    out_n = w2.shape[0] if w2 is not None else h
    in_specs = [
        pl.BlockSpec((1, NPASS, _BM, COLW),
                     lambda i: (i // nblk_half, 0, i % nblk_half, 0)),
        pl.BlockSpec((m, h), lambda i: (0, 0)),
        pl.BlockSpec((_BM, h), lambda i: (i, 0)),
        pl.BlockSpec((_BM, 1), lambda i: (i, 0)),
        pl.BlockSpec((1, h), lambda i: (0, 0)),
    ]
    args = [a_q, y, y, r, b.reshape(1, h)]
    if w2 is not None:
        in_specs.append(pl.BlockSpec((out_n, h), lambda i: (0, 0)))
        args.append(w2)
    return pl.pallas_call(
        body,
        grid=(m // _BM,),
        in_specs=in_specs,
        out_specs=pl.BlockSpec((_BM, out_n), lambda i: (i, 0)),
        out_shape=jax.ShapeDtypeStruct((m, out_n), jnp.float32),
    )(*args)


def _gram_kernel(zi_ref, zj_ref, o_ref):
    o_ref[...] = lax.dot_general(
        zi_ref[...], zj_ref[...], _DIMS_RT,
        preferred_element_type=jnp.float32)


def _gram(z):
    m, h = z.shape
    nb = m // _BM
    return pl.pallas_call(
        _gram_kernel,
        grid=(nb, nb),
        in_specs=[
            pl.BlockSpec((_BM, h), lambda i, j: (i, 0)),
            pl.BlockSpec((_BM, h), lambda i, j: (j, 0)),
        ],
        out_specs=pl.BlockSpec((_BM, _BM), lambda i, j: (i, j)),
        out_shape=jax.ShapeDtypeStruct((m, m), jnp.float32),
    )(z, z)


def kernel(edge_index, edge_weight, feat, W1, b1, W2, b2):
    src = edge_index[0]
    dst = edge_index[1]
    a_flat, deg = _build_adjacency(src, dst, edge_weight)
    a_q = a_flat.reshape(2, NPASS, HALF, COLW)
    r = (1.0 / (deg + 1.0)).reshape(N, 1)

    y1 = _matmul_rt(feat, W1)
    y2 = _conv_layer(a_q, y1, r, b1, w2=W2, relu=True)
    z = _conv_layer(a_q, y2, r, b2)
    adj_rec = _gram(z)
    return (z, adj_rec)


# speedup vs baseline: 3.4662x; 1.3165x over previous
import functools

import jax
import jax.numpy as jnp
from jax import lax
from jax.experimental import pallas as pl
from jax.experimental.pallas import tpu as pltpu
from jax.experimental.pallas import tpu_sc as plsc

N = 2048
E = 32768
H1 = 512
H2 = 128

NC, NS = 2, 16
EPT = E // NS
NCHUNK = EPT // 128
HALF = N // 2
COLW = 512
NPASS = N // COLW
PELEMS = HALF * COLW
PSTRIPE = PELEMS // NS


def _build_adjacency(src, dst, w):
    mesh = plsc.VectorSubcoreMesh(
        core_axis_name="c", subcore_axis_name="s",
        num_cores=NC, num_subcores=NS)

    @functools.partial(
        pl.kernel,
        out_type=[
            jax.ShapeDtypeStruct((2 * NPASS * PELEMS,), jnp.float32),
            jax.ShapeDtypeStruct((N,), jnp.float32),
        ],
        mesh=mesh,
        scratch_types=[
            pltpu.VMEM((EPT,), jnp.int32),
            pltpu.VMEM((EPT,), jnp.int32),
            pltpu.VMEM((EPT,), jnp.float32),
            [pltpu.VMEM((EPT,), jnp.int32) for _ in range(NPASS)],
            [pltpu.VMEM((EPT,), jnp.float32) for _ in range(NPASS)],
            pltpu.VMEM((EPT,), jnp.int32),
            pltpu.VMEM((EPT,), jnp.float32),
            pltpu.VMEM((8192,), jnp.float32),
            pltpu.VMEM((128,), jnp.float32),
            pltpu.VMEM_SHARED((PELEMS,), jnp.float32),
            pltpu.VMEM_SHARED((PELEMS,), jnp.float32),
            pltpu.VMEM_SHARED((HALF,), jnp.float32),
            pltpu.SemaphoreType.DMA,
        ],
    )
    def adj_kernel(src_h, dst_h, w_h, a_out, deg_out,
                   src_v, dst_v, w_v, idx_v, val_v, didx_v, dval_v,
                   zeros_v, junk_v, sh_a0, sh_a1, sh_d, sem):
        bufs = (sh_a0, sh_a1)
        cid = lax.axis_index("c")
        sid = lax.axis_index("s")
        base_e = sid * EPT
        loads = [
            pltpu.async_copy(src_h.at[pl.ds(base_e, EPT)], src_v, sem),
            pltpu.async_copy(dst_h.at[pl.ds(base_e, EPT)], dst_v, sem),
            pltpu.async_copy(w_h.at[pl.ds(base_e, EPT)], w_v, sem),
        ]

        def zero_body(i, carry):
            zeros_v[pl.ds(i * 16, 16)] = jnp.zeros((16,), jnp.float32)
            return carry
        lax.fori_loop(0, 8192 // 16, zero_body, 0)

        def zero_stripe(buf):
            descs = [
                pltpu.async_copy(
                    zeros_v, buf.at[pl.ds(sid * PSTRIPE + jz * 8192, 8192)], sem)
                for jz in range(PSTRIPE // 8192)]
            for d in descs:
                d.wait()

        zero_stripe(sh_a0)
        zero_stripe(sh_a1)

        @pl.when(sid == 0)
        def _zero_deg():
            pltpu.sync_copy(zeros_v.at[pl.ds(0, HALF)], sh_d)

        for d in loads:
            d.wait()
        plsc.subcore_barrier()

        row0 = cid * HALF
        for k in range(NPASS):
            buf = bufs[k % 2]
            col0 = k * COLW

            def chunk_body(j, carry):
                for v in range(8):
                    e0 = j * 128 + v * 16
                    s_ = src_v[pl.ds(e0, 16)]
                    d_ = dst_v[pl.ds(e0, 16)]
                    wv = w_v[pl.ds(e0, 16)]
                    dl = d_ - row0
                    sl = s_ - col0
                    in_row = (d_ >= row0) & (d_ < row0 + HALF)
                    m = in_row & (s_ >= col0) & (s_ < col0 + COLW)
                    dummy = lax.iota(jnp.int32, 16) + e0
                    idx_v[k][pl.ds(e0, 16)] = jnp.where(m, dl * COLW + sl, dummy)
                    val_v[k][pl.ds(e0, 16)] = jnp.where(m, wv, 0.0)
                    if k == 0:
                        didx_v[pl.ds(e0, 16)] = jnp.where(
                            in_row, dl, dummy & (HALF - 1))
                        dval_v[pl.ds(e0, 16)] = jnp.where(in_row, 1.0, 0.0)
                return carry
            lax.fori_loop(0, NCHUNK, chunk_body, 0)

            adds = [pltpu.async_copy(val_v[k], buf.at[idx_v[k]], sem,
                                     add=True)]
            if k == 0:
                adds.append(
                    pltpu.async_copy(dval_v, sh_d.at[didx_v], sem, add=True))
            for d in adds:
                d.wait()

            plsc.subcore_barrier()

            if k >= 1:
                kp = k - 1
                pbuf = bufs[kp % 2]
                qbase = (cid * NPASS + kp) * PELEMS + sid * PSTRIPE
                pltpu.sync_copy(pbuf.at[pl.ds(sid * PSTRIPE, PSTRIPE)],
                                a_out.at[pl.ds(qbase, PSTRIPE)])
                if kp + 2 < NPASS:
                    zero_stripe(pbuf)
            plsc.subcore_barrier()

        kp = NPASS - 1
        pbuf = bufs[kp % 2]
        qbase = (cid * NPASS + kp) * PELEMS + sid * PSTRIPE
        pltpu.sync_copy(pbuf.at[pl.ds(sid * PSTRIPE, PSTRIPE)],
                        a_out.at[pl.ds(qbase, PSTRIPE)])

        @pl.when(sid == 0)
        def _write_deg():
            pltpu.sync_copy(sh_d, deg_out.at[pl.ds(cid * HALF, HALF)])

    return adj_kernel(src, dst, w)


_BM = 256


def _split_bf16(x):
    hi = x.astype(jnp.bfloat16)
    lo = (x - hi.astype(jnp.float32)).astype(jnp.bfloat16)
    return hi, lo


def _dot3(x, w, dims):
    xh, xl = _split_bf16(x)
    wh, wl = _split_bf16(w)
    acc = lax.dot_general(xh, wh, dims, preferred_element_type=jnp.float32)
    acc += lax.dot_general(xh, wl, dims, preferred_element_type=jnp.float32)
    acc += lax.dot_general(xl, wh, dims, preferred_element_type=jnp.float32)
    return acc


_DIMS_RT = (((1,), (1,)), ((), ()))
_DIMS_NN = (((1,), (0,)), ((), ()))


def _mm_rt_kernel(x_ref, wh_ref, wl_ref, o_ref):
    xh, xl = _split_bf16(x_ref[...])
    acc = lax.dot_general(xh, wh_ref[...], _DIMS_RT,
                          preferred_element_type=jnp.float32)
    acc += lax.dot_general(xh, wl_ref[...], _DIMS_RT,
                           preferred_element_type=jnp.float32)
    acc += lax.dot_general(xl, wh_ref[...], _DIMS_RT,
                           preferred_element_type=jnp.float32)
    o_ref[...] = acc


def _matmul_rt(x, wh, wl):
    m, k = x.shape
    n = wh.shape[0]
    return pl.pallas_call(
        _mm_rt_kernel,
        grid=(m // _BM,),
        in_specs=[
            pl.BlockSpec((_BM, k), lambda i: (i, 0)),
            pl.BlockSpec((n, k), lambda i: (0, 0)),
            pl.BlockSpec((n, k), lambda i: (0, 0)),
        ],
        out_specs=pl.BlockSpec((_BM, n), lambda i: (i, 0)),
        out_shape=jax.ShapeDtypeStruct((m, n), jnp.float32),
    )(x, wh, wl)


def _conv_layer(a_q, y, yh, yl, r, b, w2=None, relu=False):
    m, h = y.shape
    nblk_half = HALF // _BM

    def body(a_ref, yh_ref, yl_ref, yblk_ref, r_ref, b_ref, *rest):
        if w2 is not None:
            w2_ref, o_ref = rest
        else:
            (o_ref,) = rest
        acc = None
        for s in range(NPASS):
            ah = a_ref[0, s].astype(jnp.bfloat16)
            t = lax.dot_general(ah, yh_ref[pl.ds(s * COLW, COLW), :],
                                _DIMS_NN, preferred_element_type=jnp.float32)
            t += lax.dot_general(ah, yl_ref[pl.ds(s * COLW, COLW), :],
                                 _DIMS_NN, preferred_element_type=jnp.float32)
            acc = t if acc is None else acc + t
        h_ = (acc + yblk_ref[...]) * r_ref[...] + b_ref[...]
        if relu:
            h_ = jnp.maximum(h_, 0.0)
        if w2 is not None:
            h_ = lax.dot_general(
                h_, w2_ref[...], _DIMS_RT,
                preferred_element_type=jnp.float32)
        o_ref[...] = h_

    out_n = w2.shape[0] if w2 is not None else h
    in_specs = [
        pl.BlockSpec((1, NPASS, _BM, COLW),
                     lambda i: (i // nblk_half, 0, i % nblk_half, 0)),
        pl.BlockSpec((m, h), lambda i: (0, 0)),
        pl.BlockSpec((m, h), lambda i: (0, 0)),
        pl.BlockSpec((_BM, h), lambda i: (i, 0)),
        pl.BlockSpec((_BM, 1), lambda i: (i, 0)),
        pl.BlockSpec((1, h), lambda i: (0, 0)),
    ]
    args = [a_q, yh, yl, y, r, b.reshape(1, h)]
    if w2 is not None:
        in_specs.append(pl.BlockSpec((out_n, h), lambda i: (0, 0)))
        args.append(w2)
    return pl.pallas_call(
        body,
        grid=(m // _BM,),
        in_specs=in_specs,
        out_specs=pl.BlockSpec((_BM, out_n), lambda i: (i, 0)),
        out_shape=jax.ShapeDtypeStruct((m, out_n), jnp.float32),
    )(*args)


def _gram_kernel(zi_ref, zfull_ref, o_ref):
    o_ref[...] = lax.dot_general(
        zi_ref[...], zfull_ref[...], _DIMS_RT,
        preferred_element_type=jnp.float32)


def _gram(z):
    m, h = z.shape
    return pl.pallas_call(
        _gram_kernel,
        grid=(m // _BM,),
        in_specs=[
            pl.BlockSpec((_BM, h), lambda i: (i, 0)),
            pl.BlockSpec((m, h), lambda i: (0, 0)),
        ],
        out_specs=pl.BlockSpec((_BM, m), lambda i: (i, 0)),
        out_shape=jax.ShapeDtypeStruct((m, m), jnp.float32),
    )(z, z)


def kernel(edge_index, edge_weight, feat, W1, b1, W2, b2):
    src = edge_index[0]
    dst = edge_index[1]
    a_flat, deg = _build_adjacency(src, dst, edge_weight)
    a_q = a_flat.reshape(2, NPASS, HALF, COLW)
    r = (1.0 / (deg + 1.0)).reshape(N, 1)

    w1h = W1.astype(jnp.bfloat16)
    w1l = (W1 - w1h.astype(jnp.float32)).astype(jnp.bfloat16)
    y1 = _matmul_rt(feat, w1h, w1l)
    y1h = y1.astype(jnp.bfloat16)
    y1l = (y1 - y1h.astype(jnp.float32)).astype(jnp.bfloat16)
    y2 = _conv_layer(a_q, y1, y1h, y1l, r, b1, w2=W2, relu=True)
    y2h = y2.astype(jnp.bfloat16)
    y2l = (y2 - y2h.astype(jnp.float32)).astype(jnp.bfloat16)
    z = _conv_layer(a_q, y2, y2h, y2l, r, b2)
    adj_rec = _gram(z)
    return (z, adj_rec)
